# baseline (speedup 1.0000x reference)
import functools

import jax
import jax.numpy as jnp
from jax import lax
from jax.experimental import pallas as pl
from jax.experimental.pallas import tpu as pltpu
from jax.experimental.pallas import tpu_sc as plsc

EMB = 16
NC = 2
NS = 16
NW = NC * NS
CHUNK = 1024


def _edge_encode(n_edges):
    n_chunks = n_edges // CHUNK
    full, extra = divmod(n_chunks, NW)

    mesh = plsc.VectorSubcoreMesh(core_axis_name="c", subcore_axis_name="s")

    @functools.partial(
        pl.kernel,
        mesh=mesh,
        out_type=jax.ShapeDtypeStruct((n_edges * EMB // 128, 128), jnp.float32),
        compiler_params=pltpu.CompilerParams(
            needs_layout_passes=False, use_tc_tiling_on_sc=False
        ),
        scratch_types=[
            pltpu.VMEM((CHUNK * 2 // 128, 128), jnp.int32),
            pltpu.VMEM((CHUNK * EMB // 128, 128), jnp.float32),
            pltpu.VMEM((4, 128), jnp.float32),
            pltpu.VMEM((4 * EMB,), jnp.float32),
        ],
    )
    def run(attr_hbm, w_hbm, out_hbm, attr_v, rows_v, w_v, lut_v):
        wid = lax.axis_index("s") * NC + lax.axis_index("c")
        iota = lax.iota(jnp.int32, 16)

        pltpu.sync_copy(w_hbm, w_v)
        wt0 = w_v[0, pl.ds(0, 16)]
        wt1 = w_v[1, pl.ds(0, 16)]
        wd0 = w_v[2, pl.ds(0, 16)]
        wd1 = w_v[3, pl.ds(0, 16)]
        lut_v[pl.ds(0, 16)] = wt0 + wd0
        lut_v[pl.ds(16, 16)] = wt0 + wd1
        lut_v[pl.ds(32, 16)] = wt1 + wd0
        lut_v[pl.ds(48, 16)] = wt1 + wd1

        n_mine = full + jnp.where(wid < extra, 1, 0)

        def chunk_body(t, carry):
            ki = wid + t * NW
            pltpu.sync_copy(
                attr_hbm.at[pl.ds(ki * (CHUNK * 2 // 128), CHUNK * 2 // 128), :],
                attr_v,
            )

            def group_body(g, c2):
                e = g * 16 + iota
                a0 = plsc.load_gather(attr_v, [e >> 6, (e & 63) * 2])
                a1 = plsc.load_gather(attr_v, [e >> 6, (e & 63) * 2 + 1])
                cb = (a0 * 2 + a1) * 16
                r = e >> 3
                c0 = (e & 7) * 16
                for d in range(16):
                    pm = jnp.bitwise_and(iota + d, 15)
                    val = plsc.load_gather(lut_v, [cb + pm])
                    plsc.store_scatter(rows_v, [r, c0 + pm], val)
                return c2

            lax.fori_loop(0, CHUNK // 16, group_body, 0)
            pltpu.sync_copy(
                rows_v,
                out_hbm.at[pl.ds(ki * (CHUNK * EMB // 128), CHUNK * EMB // 128), :],
            )
            return carry

        lax.fori_loop(0, n_mine, chunk_body, 0)

    return run


def kernel(edge_attr, W_type, W_dir):
    n_edges = edge_attr.shape[0]
    run = _edge_encode(n_edges)
    w = jnp.concatenate([W_type, W_dir], axis=0)
    w = jnp.pad(w, ((0, 0), (0, 128 - EMB)))
    attr = edge_attr.astype(jnp.int32).reshape(n_edges * 2 // 128, 128)
    out = run(attr, w)
    return out.reshape(n_edges, EMB)
